# baseline (speedup 1.0000x reference)
import functools

import jax
import jax.numpy as jnp
from jax import lax
from jax.experimental import pallas as pl
from jax.experimental.pallas import tpu as pltpu
from jax.experimental.pallas import tpu_sc as plsc

N = 10000
NPAD = 10240
E = 320000
EBA = 2048
NBLKA = 157
EPAD = EBA * NBLKA
EPADB = EPAD + 256
F = 640
NC, NS, L = 2, 16, 16
NW = NC * NS
CH = 80
NCHUNK = NPAD // CH
CPW = NCHUNK // NW
GW = 32
SB = 1024
STG = 272
R = 512
PACK = 16384

_mesh = lambda: plsc.VectorSubcoreMesh(core_axis_name="c", subcore_axis_name="s")
_sc_params = pltpu.CompilerParams(needs_layout_passes=False)


@functools.partial(
    pl.kernel,
    mesh=_mesh(),
    compiler_params=_sc_params,
    out_type=[
        jax.ShapeDtypeStruct((NPAD,), jnp.float32),
        jax.ShapeDtypeStruct((NCHUNK * EPADB,), jnp.int32),
        jax.ShapeDtypeStruct((NCHUNK * L,), jnp.int32),
    ],
    scratch_types=[
        pltpu.VMEM((EBA,), jnp.int32),
        pltpu.VMEM((EBA,), jnp.int32),
        pltpu.VMEM((EBA,), jnp.int32),
        pltpu.VMEM((EBA,), jnp.int32),
        pltpu.VMEM((CPW * CH,), jnp.float32),
        pltpu.VMEM((CPW * STG,), jnp.int32),
        pltpu.VMEM((L,), jnp.int32),
        pltpu.SemaphoreType.DMA,
        pltpu.SemaphoreType.DMA,
    ],
)
def _bucket_kernel(dst_hbm, src_hbm, deg_hbm, bkt_hbm, cnt_hbm,
                   dstb0, srcb0, dstb1, srcb1, degl, stage, crow, semA, semB):
    wid = lax.axis_index("s") * NC + lax.axis_index("c")
    c0id = wid * CPW
    b0 = c0id * CH
    ones = jnp.full((L,), 1.0, jnp.float32)
    zc = jnp.zeros((L,), jnp.int32)

    def init(i, c):
        degl[pl.ds(i * L, L)] = ones
        return c

    lax.fori_loop(0, CPW * CH // L, init, 0)

    def issue(b):
        @pl.when((b % 2 == 0) & (b < NBLKA))
        def _():
            pltpu.async_copy(dst_hbm.at[pl.ds(b * EBA, EBA)], dstb0, semA)
            pltpu.async_copy(src_hbm.at[pl.ds(b * EBA, EBA)], srcb0, semA)

        @pl.when((b % 2 == 1) & (b < NBLKA))
        def _():
            pltpu.async_copy(dst_hbm.at[pl.ds(b * EBA, EBA)], dstb1, semB)
            pltpu.async_copy(src_hbm.at[pl.ds(b * EBA, EBA)], srcb1, semB)

    def waitblk(b):
        @pl.when(b % 2 == 0)
        def _():
            pltpu.make_async_copy(dst_hbm.at[pl.ds(0, EBA)], dstb0, semA).wait()
            pltpu.make_async_copy(src_hbm.at[pl.ds(0, EBA)], srcb0, semA).wait()

        @pl.when(b % 2 == 1)
        def _():
            pltpu.make_async_copy(dst_hbm.at[pl.ds(0, EBA)], dstb1, semB).wait()
            pltpu.make_async_copy(src_hbm.at[pl.ds(0, EBA)], srcb1, semB).wait()

    issue(jnp.int32(0))

    def blk(b, carry):
        issue(b + 1)
        waitblk(b)
        even = (b % 2) == 0

        def grp(i, carry2):
            cs = list(carry2[0:CPW])
            fs = list(carry2[CPW:2 * CPW])
            d0 = dstb0[pl.ds(i * L, L)]
            d1 = dstb1[pl.ds(i * L, L)]
            s0 = srcb0[pl.ds(i * L, L)]
            s1 = srcb1[pl.ds(i * L, L)]
            d = jnp.where(even, d0, d1)
            s = jnp.where(even, s0, s1)
            ld = d - b0
            plsc.addupdate_scatter(degl, [ld], ones,
                                   mask=(ld >= 0) & (ld < CPW * CH))
            for q in range(CPW):
                ldq = ld - q * CH
                mq = (ldq >= 0) & (ldq < CH)
                pkq = (ldq * PACK) | s
                posq = cs[q] + jnp.cumsum(mq.astype(jnp.int32)) - 1 + q * STG
                plsc.store_scatter(stage, [posq], pkq, mask=mq)
                cs[q] = cs[q] + plsc.all_reduce_population_count(mq)
                prq = jnp.any(cs[q] >= 256)

                @pl.when(prq)
                def _(q=q, fq=fs[q]):
                    pltpu.sync_copy(
                        stage.at[pl.ds(q * STG, 256)],
                        bkt_hbm.at[pl.ds((c0id + q) * EPADB + fq * 256, 256)])
                    stage[pl.ds(q * STG, L)] = stage[pl.ds(q * STG + 256, L)]

                fs[q] = fs[q] + prq.astype(jnp.int32)
                cs[q] = jnp.where(cs[q] >= 256, cs[q] - 256, cs[q])
            return tuple(cs) + tuple(fs)

        return lax.fori_loop(0, EBA // L, grp, carry)

    carry0 = (zc, zc, zc, zc, 0, 0, 0, 0)
    carry = lax.fori_loop(0, NBLKA, blk, carry0)
    for q in range(CPW):
        cq, fq = carry[q], carry[CPW + q]
        pltpu.sync_copy(stage.at[pl.ds(q * STG, 256)],
                        bkt_hbm.at[pl.ds((c0id + q) * EPADB + fq * 256, 256)])
        crow[pl.ds(0, L)] = fq * 256 + cq
        pltpu.sync_copy(crow, cnt_hbm.at[pl.ds((c0id + q) * L, L)])
    pltpu.sync_copy(degl, deg_hbm.at[pl.ds(b0, CPW * CH)])


@functools.partial(
    pl.kernel,
    mesh=_mesh(),
    compiler_params=_sc_params,
    out_type=jax.ShapeDtypeStruct((NPAD, F), jnp.float32),
    scratch_types=[
        pltpu.VMEM((CH + 1, F), jnp.float32),
        pltpu.VMEM((GW, F), jnp.float32),
        pltpu.VMEM((GW, F), jnp.float32),
        pltpu.VMEM((SB,), jnp.int32),
        pltpu.VMEM((GW,), jnp.int32),
        pltpu.VMEM((GW,), jnp.int32),
        pltpu.VMEM((L,), jnp.int32),
        pltpu.SemaphoreType.DMA,
        pltpu.SemaphoreType.DMA,
    ],
)
def _segmax_kernel(g_hbm, bkt_hbm, cnt_hbm, out_hbm,
                   acc, rows0, rows1, stage, fidx0, fidx1, crow, sem0, sem1):
    wid = lax.axis_index("s") * NC + lax.axis_index("c")
    lanes = lax.iota(jnp.int32, L)
    neg1 = jnp.full((L,), -1, jnp.int32)
    GPB = SB // GW

    def process(rows_ref, ld_a, ld_b):
        def row_body(j, c):
            jv = lanes * 0 + j
            sl = jnp.where(jv < L, ld_a, ld_b)
            lj = jnp.max(jnp.where(lanes == j % L, sl, neg1))
            for f in range(F // L):
                a = acc[lj, pl.ds(f * L, L)]
                r = rows_ref[j, pl.ds(f * L, L)]
                acc[lj, pl.ds(f * L, L)] = jnp.maximum(a, r)
            return c

        lax.fori_loop(0, GW, row_body, 0)

    for p in range(CPW):
        chunk = wid * CPW + p
        base = chunk * CH
        pltpu.sync_copy(g_hbm.at[pl.ds(base, CH)], acc.at[pl.ds(0, CH)])
        pltpu.sync_copy(cnt_hbm.at[pl.ds(chunk * L, L)], crow)
        total = jnp.max(crow[pl.ds(0, L)])
        ngrp = (total + GW - 1) // GW
        pltpu.sync_copy(bkt_hbm.at[pl.ds(chunk * EPADB, SB)], stage)

        def unpack(word_off, grp_idx):
            pka = stage[pl.ds(word_off, L)]
            pkb = stage[pl.ds(word_off + L, L)]
            va = (grp_idx * GW + lanes) < total
            vb = (grp_idx * GW + L + lanes) < total
            sa = jnp.where(va, pka & (PACK - 1), 0)
            sb = jnp.where(vb, pkb & (PACK - 1), 0)
            la = jnp.where(va, pka // PACK, CH)
            lb = jnp.where(vb, pkb // PACK, CH)
            return sa, sb, la, lb

        sa0, sb0, la0, lb0 = unpack(0, jnp.int32(0))

        @pl.when(ngrp > 0)
        def _():
            fidx0[pl.ds(0, L)] = sa0
            fidx0[pl.ds(L, L)] = sb0
            pltpu.async_copy(g_hbm.at[fidx0], rows0, sem0)

        def grp_body(gi, ld_cur):
            ld_a, ld_b = ld_cur
            nxt = gi + 1
            par = gi % 2

            @pl.when((nxt % GPB == 0) & (nxt < ngrp))
            def _():
                pltpu.sync_copy(
                    bkt_hbm.at[pl.ds(chunk * EPADB + (nxt // GPB) * SB, SB)],
                    stage)

            sa, sb, la, lb = unpack((nxt % GPB) * GW, nxt)

            @pl.when((par == 0) & (nxt < ngrp))
            def _():
                fidx1[pl.ds(0, L)] = sa
                fidx1[pl.ds(L, L)] = sb
                pltpu.async_copy(g_hbm.at[fidx1], rows1, sem1)

            @pl.when((par == 1) & (nxt < ngrp))
            def _():
                fidx0[pl.ds(0, L)] = sa
                fidx0[pl.ds(L, L)] = sb
                pltpu.async_copy(g_hbm.at[fidx0], rows0, sem0)

            @pl.when(par == 0)
            def _():
                pltpu.make_async_copy(g_hbm.at[fidx0], rows0, sem0).wait()
                process(rows0, ld_a, ld_b)

            @pl.when(par == 1)
            def _():
                pltpu.make_async_copy(g_hbm.at[fidx1], rows1, sem1).wait()
                process(rows1, ld_a, ld_b)

            return (la, lb)

        lax.fori_loop(0, ngrp, grp_body, (la0, lb0))
        pltpu.sync_copy(acc.at[pl.ds(0, CH)], out_hbm.at[pl.ds(base, CH)])


def _tc_g1_body(deg_ref, x_ref, w_ref, g_ref, dinv_ref):
    dinv = lax.rsqrt(jnp.maximum(deg_ref[...], 1e-12))
    dinv_ref[...] = dinv
    h = jnp.dot(x_ref[...], w_ref[...], preferred_element_type=jnp.float32)
    g_ref[...] = dinv * h


def _tc_mid_body(m_ref, dinv_ref, b1_ref, w2_ref, id_ref, g2_ref):
    dinv = dinv_ref[...]
    idv = jnp.maximum(dinv * m_ref[...] + b1_ref[...], 0.0)
    id_ref[...] = idv
    h2 = jnp.dot(idv, w2_ref[...], preferred_element_type=jnp.float32)
    g2_ref[...] = dinv * h2


def _tc_head_body(m_ref, dinv_ref, b2_ref, id_ref, w3a_ref, w3b_ref, b3_ref, o_ref):
    h2 = jnp.maximum(dinv_ref[...] * m_ref[...] + b2_ref[...], 0.0)
    o_ref[...] = (
        jnp.dot(h2, w3a_ref[...], preferred_element_type=jnp.float32)
        + jnp.dot(id_ref[...], w3b_ref[...], preferred_element_type=jnp.float32)
        + b3_ref[...]
    )


def kernel(x, edge_index, W1, b1, W2, b2, W3, b3):
    src = edge_index[0]
    dst = edge_index[1]
    srcp = jnp.pad(src, (0, EPAD - E))
    dstp = jnp.pad(dst, (0, EPAD - E), constant_values=-1)

    deg, bkt, cnts = _bucket_kernel(dstp, srcp)
    deg2 = deg.reshape(NPAD, 1)

    xp = jnp.pad(x, ((0, NPAD - N), (0, 8 - x.shape[1])))
    W1p = jnp.pad(W1, ((0, 8 - W1.shape[0]), (0, 0)))

    g1, dinv2 = pl.pallas_call(
        _tc_g1_body,
        grid=(NPAD // R,),
        in_specs=[
            pl.BlockSpec((R, 1), lambda i: (i, 0)),
            pl.BlockSpec((R, 8), lambda i: (i, 0)),
            pl.BlockSpec((8, F), lambda i: (0, 0)),
        ],
        out_specs=[
            pl.BlockSpec((R, F), lambda i: (i, 0)),
            pl.BlockSpec((R, 1), lambda i: (i, 0)),
        ],
        out_shape=[
            jax.ShapeDtypeStruct((NPAD, F), jnp.float32),
            jax.ShapeDtypeStruct((NPAD, 1), jnp.float32),
        ],
    )(deg2, xp, W1p)

    m1 = _segmax_kernel(g1, bkt, cnts)

    identity, g2 = pl.pallas_call(
        _tc_mid_body,
        grid=(NPAD // R,),
        in_specs=[
            pl.BlockSpec((R, F), lambda i: (i, 0)),
            pl.BlockSpec((R, 1), lambda i: (i, 0)),
            pl.BlockSpec((1, F), lambda i: (0, 0)),
            pl.BlockSpec((F, F), lambda i: (0, 0)),
        ],
        out_specs=[
            pl.BlockSpec((R, F), lambda i: (i, 0)),
            pl.BlockSpec((R, F), lambda i: (i, 0)),
        ],
        out_shape=[
            jax.ShapeDtypeStruct((NPAD, F), jnp.float32),
            jax.ShapeDtypeStruct((NPAD, F), jnp.float32),
        ],
    )(m1, dinv2, b1.reshape(1, F), W2)

    m2 = _segmax_kernel(g2, bkt, cnts)

    D_OUT = W3.shape[1]
    W3p = jnp.pad(W3, ((0, 0), (0, 128 - D_OUT)))
    b3p = jnp.pad(b3, (0, 128 - D_OUT)).reshape(1, 128)

    out = pl.pallas_call(
        _tc_head_body,
        grid=(NPAD // R,),
        in_specs=[
            pl.BlockSpec((R, F), lambda i: (i, 0)),
            pl.BlockSpec((R, 1), lambda i: (i, 0)),
            pl.BlockSpec((1, F), lambda i: (0, 0)),
            pl.BlockSpec((R, F), lambda i: (i, 0)),
            pl.BlockSpec((F, 128), lambda i: (0, 0)),
            pl.BlockSpec((F, 128), lambda i: (0, 0)),
            pl.BlockSpec((1, 128), lambda i: (0, 0)),
        ],
        out_specs=pl.BlockSpec((R, 128), lambda i: (i, 0)),
        out_shape=jax.ShapeDtypeStruct((NPAD, 128), jnp.float32),
    )(m2, dinv2, b2.reshape(1, F), identity, W3p[:F], W3p[F:], b3p)

    return out[:N, :D_OUT]


# speedup vs baseline: 1.4266x; 1.4266x over previous
import functools

import jax
import jax.numpy as jnp
from jax import lax
from jax.experimental import pallas as pl
from jax.experimental.pallas import tpu as pltpu
from jax.experimental.pallas import tpu_sc as plsc

N = 10000
NPAD = 10240
E = 320000
EBA = 2048
NBLKA = 157
EPAD = EBA * NBLKA
EPADB = EPAD + 256
F = 640
NC, NS, L = 2, 16, 16
NW = NC * NS
CH = 80
NCHUNK = NPAD // CH
CPW = NCHUNK // NW
GW = 32
SB = 1024
STG = 272
R = 512
PACK = 16384

_mesh = lambda: plsc.VectorSubcoreMesh(core_axis_name="c", subcore_axis_name="s")
_sc_params = pltpu.CompilerParams(needs_layout_passes=False)


@functools.partial(
    pl.kernel,
    mesh=_mesh(),
    compiler_params=_sc_params,
    out_type=[
        jax.ShapeDtypeStruct((NPAD,), jnp.float32),
        jax.ShapeDtypeStruct((NCHUNK * EPADB,), jnp.int32),
        jax.ShapeDtypeStruct((NCHUNK * L,), jnp.int32),
    ],
    scratch_types=[
        pltpu.VMEM((EBA,), jnp.int32),
        pltpu.VMEM((EBA,), jnp.int32),
        pltpu.VMEM((EBA,), jnp.int32),
        pltpu.VMEM((EBA,), jnp.int32),
        pltpu.VMEM((CPW * CH,), jnp.float32),
        pltpu.VMEM((CPW * STG,), jnp.int32),
        pltpu.VMEM((L,), jnp.int32),
        pltpu.SemaphoreType.DMA,
        pltpu.SemaphoreType.DMA,
    ],
)
def _bucket_kernel(dst_hbm, src_hbm, deg_hbm, bkt_hbm, cnt_hbm,
                   dstb0, srcb0, dstb1, srcb1, degl, stage, crow, semA, semB):
    wid = lax.axis_index("s") * NC + lax.axis_index("c")
    c0id = wid * CPW
    b0 = c0id * CH
    ones = jnp.full((L,), 1.0, jnp.float32)
    zc = jnp.zeros((L,), jnp.int32)

    def init(i, c):
        degl[pl.ds(i * L, L)] = ones
        return c

    lax.fori_loop(0, CPW * CH // L, init, 0)

    def issue(b):
        @pl.when((b % 2 == 0) & (b < NBLKA))
        def _():
            pltpu.async_copy(dst_hbm.at[pl.ds(b * EBA, EBA)], dstb0, semA)
            pltpu.async_copy(src_hbm.at[pl.ds(b * EBA, EBA)], srcb0, semA)

        @pl.when((b % 2 == 1) & (b < NBLKA))
        def _():
            pltpu.async_copy(dst_hbm.at[pl.ds(b * EBA, EBA)], dstb1, semB)
            pltpu.async_copy(src_hbm.at[pl.ds(b * EBA, EBA)], srcb1, semB)

    def waitblk(b):
        @pl.when(b % 2 == 0)
        def _():
            pltpu.make_async_copy(dst_hbm.at[pl.ds(0, EBA)], dstb0, semA).wait()
            pltpu.make_async_copy(src_hbm.at[pl.ds(0, EBA)], srcb0, semA).wait()

        @pl.when(b % 2 == 1)
        def _():
            pltpu.make_async_copy(dst_hbm.at[pl.ds(0, EBA)], dstb1, semB).wait()
            pltpu.make_async_copy(src_hbm.at[pl.ds(0, EBA)], srcb1, semB).wait()

    issue(jnp.int32(0))

    def blk(b, carry):
        issue(b + 1)
        waitblk(b)
        even = (b % 2) == 0

        def grp(i, carry2):
            cs = list(carry2[0:CPW])
            fs = list(carry2[CPW:2 * CPW])
            d0 = dstb0[pl.ds(i * L, L)]
            d1 = dstb1[pl.ds(i * L, L)]
            s0 = srcb0[pl.ds(i * L, L)]
            s1 = srcb1[pl.ds(i * L, L)]
            d = jnp.where(even, d0, d1)
            s = jnp.where(even, s0, s1)
            ld = d - b0
            plsc.addupdate_scatter(degl, [ld], ones,
                                   mask=(ld >= 0) & (ld < CPW * CH))
            for q in range(CPW):
                ldq = ld - q * CH
                mq = (ldq >= 0) & (ldq < CH)
                pkq = (ldq * PACK) | s
                posq = cs[q] + jnp.cumsum(mq.astype(jnp.int32)) - 1 + q * STG
                plsc.store_scatter(stage, [posq], pkq, mask=mq)
                cs[q] = cs[q] + plsc.all_reduce_population_count(mq)
                prq = jnp.any(cs[q] >= 256)

                @pl.when(prq)
                def _(q=q, fq=fs[q]):
                    pltpu.sync_copy(
                        stage.at[pl.ds(q * STG, 256)],
                        bkt_hbm.at[pl.ds((c0id + q) * EPADB + fq * 256, 256)])
                    stage[pl.ds(q * STG, L)] = stage[pl.ds(q * STG + 256, L)]

                fs[q] = fs[q] + prq.astype(jnp.int32)
                cs[q] = jnp.where(cs[q] >= 256, cs[q] - 256, cs[q])
            return tuple(cs) + tuple(fs)

        return lax.fori_loop(0, EBA // L, grp, carry, unroll=8)

    carry0 = (zc, zc, zc, zc, 0, 0, 0, 0)
    carry = lax.fori_loop(0, NBLKA, blk, carry0)
    for q in range(CPW):
        cq, fq = carry[q], carry[CPW + q]
        pltpu.sync_copy(stage.at[pl.ds(q * STG, 256)],
                        bkt_hbm.at[pl.ds((c0id + q) * EPADB + fq * 256, 256)])
        crow[pl.ds(0, L)] = fq * 256 + cq
        pltpu.sync_copy(crow, cnt_hbm.at[pl.ds((c0id + q) * L, L)])
    pltpu.sync_copy(degl, deg_hbm.at[pl.ds(b0, CPW * CH)])


@functools.partial(
    pl.kernel,
    mesh=_mesh(),
    compiler_params=_sc_params,
    out_type=jax.ShapeDtypeStruct((NPAD, F), jnp.float32),
    scratch_types=[
        pltpu.VMEM((CH + 1, F), jnp.float32),
        pltpu.VMEM((GW, F), jnp.float32),
        pltpu.VMEM((GW, F), jnp.float32),
        pltpu.VMEM((SB,), jnp.int32),
        pltpu.VMEM((GW,), jnp.int32),
        pltpu.VMEM((GW,), jnp.int32),
        pltpu.VMEM((L,), jnp.int32),
        pltpu.SemaphoreType.DMA,
        pltpu.SemaphoreType.DMA,
    ],
)
def _segmax_kernel(g_hbm, bkt_hbm, cnt_hbm, out_hbm,
                   acc, rows0, rows1, stage, fidx0, fidx1, crow, sem0, sem1):
    wid = lax.axis_index("s") * NC + lax.axis_index("c")
    lanes = lax.iota(jnp.int32, L)
    neg1 = jnp.full((L,), -1, jnp.int32)
    GPB = SB // GW

    def process(rows_ref, ld_a, ld_b):
        def row_body(j, c):
            jv = lanes * 0 + j
            sl = jnp.where(jv < L, ld_a, ld_b)
            lj = jnp.max(jnp.where(lanes == j % L, sl, neg1))
            for f0 in range(0, F // L, 8):
                avs = [acc[lj, pl.ds((f0 + k) * L, L)] for k in range(8)]
                rvs = [rows_ref[j, pl.ds((f0 + k) * L, L)] for k in range(8)]
                for k in range(8):
                    acc[lj, pl.ds((f0 + k) * L, L)] = jnp.maximum(avs[k], rvs[k])
            return c

        lax.fori_loop(0, GW, row_body, 0)

    for p in range(CPW):
        chunk = wid * CPW + p
        base = chunk * CH
        pltpu.sync_copy(g_hbm.at[pl.ds(base, CH)], acc.at[pl.ds(0, CH)])
        pltpu.sync_copy(cnt_hbm.at[pl.ds(chunk * L, L)], crow)
        total = jnp.max(crow[pl.ds(0, L)])
        ngrp = (total + GW - 1) // GW
        pltpu.sync_copy(bkt_hbm.at[pl.ds(chunk * EPADB, SB)], stage)

        def unpack(word_off, grp_idx):
            pka = stage[pl.ds(word_off, L)]
            pkb = stage[pl.ds(word_off + L, L)]
            va = (grp_idx * GW + lanes) < total
            vb = (grp_idx * GW + L + lanes) < total
            sa = jnp.where(va, pka & (PACK - 1), 0)
            sb = jnp.where(vb, pkb & (PACK - 1), 0)
            la = jnp.where(va, pka // PACK, CH)
            lb = jnp.where(vb, pkb // PACK, CH)
            return sa, sb, la, lb

        sa0, sb0, la0, lb0 = unpack(0, jnp.int32(0))

        @pl.when(ngrp > 0)
        def _():
            fidx0[pl.ds(0, L)] = sa0
            fidx0[pl.ds(L, L)] = sb0
            pltpu.async_copy(g_hbm.at[fidx0], rows0, sem0)

        def grp_body(gi, ld_cur):
            ld_a, ld_b = ld_cur
            nxt = gi + 1
            par = gi % 2

            @pl.when((nxt % GPB == 0) & (nxt < ngrp))
            def _():
                pltpu.sync_copy(
                    bkt_hbm.at[pl.ds(chunk * EPADB + (nxt // GPB) * SB, SB)],
                    stage)

            sa, sb, la, lb = unpack((nxt % GPB) * GW, nxt)

            @pl.when((par == 0) & (nxt < ngrp))
            def _():
                fidx1[pl.ds(0, L)] = sa
                fidx1[pl.ds(L, L)] = sb
                pltpu.async_copy(g_hbm.at[fidx1], rows1, sem1)

            @pl.when((par == 1) & (nxt < ngrp))
            def _():
                fidx0[pl.ds(0, L)] = sa
                fidx0[pl.ds(L, L)] = sb
                pltpu.async_copy(g_hbm.at[fidx0], rows0, sem0)

            @pl.when(par == 0)
            def _():
                pltpu.make_async_copy(g_hbm.at[fidx0], rows0, sem0).wait()
                process(rows0, ld_a, ld_b)

            @pl.when(par == 1)
            def _():
                pltpu.make_async_copy(g_hbm.at[fidx1], rows1, sem1).wait()
                process(rows1, ld_a, ld_b)

            return (la, lb)

        lax.fori_loop(0, ngrp, grp_body, (la0, lb0))
        pltpu.sync_copy(acc.at[pl.ds(0, CH)], out_hbm.at[pl.ds(base, CH)])


def _tc_g1_body(deg_ref, x_ref, w_ref, g_ref, dinv_ref):
    dinv = lax.rsqrt(jnp.maximum(deg_ref[...], 1e-12))
    dinv_ref[...] = dinv
    h = jnp.dot(x_ref[...], w_ref[...], preferred_element_type=jnp.float32)
    g_ref[...] = dinv * h


def _tc_mid_body(m_ref, dinv_ref, b1_ref, w2_ref, id_ref, g2_ref):
    dinv = dinv_ref[...]
    idv = jnp.maximum(dinv * m_ref[...] + b1_ref[...], 0.0)
    id_ref[...] = idv
    h2 = jnp.dot(idv, w2_ref[...], preferred_element_type=jnp.float32)
    g2_ref[...] = dinv * h2


def _tc_head_body(m_ref, dinv_ref, b2_ref, id_ref, w3a_ref, w3b_ref, b3_ref, o_ref):
    h2 = jnp.maximum(dinv_ref[...] * m_ref[...] + b2_ref[...], 0.0)
    o_ref[...] = (
        jnp.dot(h2, w3a_ref[...], preferred_element_type=jnp.float32)
        + jnp.dot(id_ref[...], w3b_ref[...], preferred_element_type=jnp.float32)
        + b3_ref[...]
    )


def kernel(x, edge_index, W1, b1, W2, b2, W3, b3):
    src = edge_index[0]
    dst = edge_index[1]
    srcp = jnp.pad(src, (0, EPAD - E))
    dstp = jnp.pad(dst, (0, EPAD - E), constant_values=-1)

    deg, bkt, cnts = _bucket_kernel(dstp, srcp)
    deg2 = deg.reshape(NPAD, 1)

    xp = jnp.pad(x, ((0, NPAD - N), (0, 8 - x.shape[1])))
    W1p = jnp.pad(W1, ((0, 8 - W1.shape[0]), (0, 0)))

    g1, dinv2 = pl.pallas_call(
        _tc_g1_body,
        grid=(NPAD // R,),
        in_specs=[
            pl.BlockSpec((R, 1), lambda i: (i, 0)),
            pl.BlockSpec((R, 8), lambda i: (i, 0)),
            pl.BlockSpec((8, F), lambda i: (0, 0)),
        ],
        out_specs=[
            pl.BlockSpec((R, F), lambda i: (i, 0)),
            pl.BlockSpec((R, 1), lambda i: (i, 0)),
        ],
        out_shape=[
            jax.ShapeDtypeStruct((NPAD, F), jnp.float32),
            jax.ShapeDtypeStruct((NPAD, 1), jnp.float32),
        ],
    )(deg2, xp, W1p)

    m1 = _segmax_kernel(g1, bkt, cnts)

    identity, g2 = pl.pallas_call(
        _tc_mid_body,
        grid=(NPAD // R,),
        in_specs=[
            pl.BlockSpec((R, F), lambda i: (i, 0)),
            pl.BlockSpec((R, 1), lambda i: (i, 0)),
            pl.BlockSpec((1, F), lambda i: (0, 0)),
            pl.BlockSpec((F, F), lambda i: (0, 0)),
        ],
        out_specs=[
            pl.BlockSpec((R, F), lambda i: (i, 0)),
            pl.BlockSpec((R, F), lambda i: (i, 0)),
        ],
        out_shape=[
            jax.ShapeDtypeStruct((NPAD, F), jnp.float32),
            jax.ShapeDtypeStruct((NPAD, F), jnp.float32),
        ],
    )(m1, dinv2, b1.reshape(1, F), W2)

    m2 = _segmax_kernel(g2, bkt, cnts)

    D_OUT = W3.shape[1]
    W3p = jnp.pad(W3, ((0, 0), (0, 128 - D_OUT)))
    b3p = jnp.pad(b3, (0, 128 - D_OUT)).reshape(1, 128)

    out = pl.pallas_call(
        _tc_head_body,
        grid=(NPAD // R,),
        in_specs=[
            pl.BlockSpec((R, F), lambda i: (i, 0)),
            pl.BlockSpec((R, 1), lambda i: (i, 0)),
            pl.BlockSpec((1, F), lambda i: (0, 0)),
            pl.BlockSpec((R, F), lambda i: (i, 0)),
            pl.BlockSpec((F, 128), lambda i: (0, 0)),
            pl.BlockSpec((F, 128), lambda i: (0, 0)),
            pl.BlockSpec((1, 128), lambda i: (0, 0)),
        ],
        out_specs=pl.BlockSpec((R, 128), lambda i: (i, 0)),
        out_shape=jax.ShapeDtypeStruct((NPAD, 128), jnp.float32),
    )(m2, dinv2, b2.reshape(1, F), identity, W3p[:F], W3p[F:], b3p)

    return out[:N, :D_OUT]


# speedup vs baseline: 1.8595x; 1.3034x over previous
import functools

import jax
import jax.numpy as jnp
from jax import lax
from jax.experimental import pallas as pl
from jax.experimental.pallas import tpu as pltpu
from jax.experimental.pallas import tpu_sc as plsc

N = 10000
NPAD = 10240
E = 320000
EBA = 2048
NBLKA = 157
EPAD = EBA * NBLKA
EPADB = EPAD + 256
F = 640
NC, NS, L = 2, 16, 16
NW = NC * NS
CH = 80
NCHUNK = NPAD // CH
CPW = NCHUNK // NW
GW = 32
SB = 1024
STG = 272
R = 512
PACK = 16384

_mesh = lambda: plsc.VectorSubcoreMesh(core_axis_name="c", subcore_axis_name="s")
_sc_params = pltpu.CompilerParams(needs_layout_passes=False)


@functools.partial(
    pl.kernel,
    mesh=_mesh(),
    compiler_params=_sc_params,
    out_type=[
        jax.ShapeDtypeStruct((NPAD,), jnp.float32),
        jax.ShapeDtypeStruct((NCHUNK * EPADB,), jnp.int32),
        jax.ShapeDtypeStruct((NCHUNK * L,), jnp.int32),
    ],
    scratch_types=[
        pltpu.VMEM((EBA,), jnp.int32),
        pltpu.VMEM((EBA,), jnp.int32),
        pltpu.VMEM((EBA,), jnp.int32),
        pltpu.VMEM((EBA,), jnp.int32),
        pltpu.VMEM((CPW * CH,), jnp.float32),
        pltpu.VMEM((CPW * STG,), jnp.int32),
        pltpu.VMEM((L,), jnp.int32),
        pltpu.SemaphoreType.DMA,
        pltpu.SemaphoreType.DMA,
    ],
)
def _bucket_kernel(dst_hbm, src_hbm, deg_hbm, bkt_hbm, cnt_hbm,
                   dstb0, srcb0, dstb1, srcb1, degl, stage, crow, semA, semB):
    wid = lax.axis_index("s") * NC + lax.axis_index("c")
    c0id = wid * CPW
    b0 = c0id * CH
    ones = jnp.full((L,), 1.0, jnp.float32)
    zc = jnp.zeros((L,), jnp.int32)

    def init(i, c):
        degl[pl.ds(i * L, L)] = ones
        return c

    lax.fori_loop(0, CPW * CH // L, init, 0)

    def issue(b):
        @pl.when((b % 2 == 0) & (b < NBLKA))
        def _():
            pltpu.async_copy(dst_hbm.at[pl.ds(b * EBA, EBA)], dstb0, semA)
            pltpu.async_copy(src_hbm.at[pl.ds(b * EBA, EBA)], srcb0, semA)

        @pl.when((b % 2 == 1) & (b < NBLKA))
        def _():
            pltpu.async_copy(dst_hbm.at[pl.ds(b * EBA, EBA)], dstb1, semB)
            pltpu.async_copy(src_hbm.at[pl.ds(b * EBA, EBA)], srcb1, semB)

    def waitblk(b):
        @pl.when(b % 2 == 0)
        def _():
            pltpu.make_async_copy(dst_hbm.at[pl.ds(0, EBA)], dstb0, semA).wait()
            pltpu.make_async_copy(src_hbm.at[pl.ds(0, EBA)], srcb0, semA).wait()

        @pl.when(b % 2 == 1)
        def _():
            pltpu.make_async_copy(dst_hbm.at[pl.ds(0, EBA)], dstb1, semB).wait()
            pltpu.make_async_copy(src_hbm.at[pl.ds(0, EBA)], srcb1, semB).wait()

    issue(jnp.int32(0))

    def blk(b, carry):
        issue(b + 1)
        waitblk(b)
        even = (b % 2) == 0

        def grp(i, carry2):
            cs = list(carry2[0:CPW])
            fs = list(carry2[CPW:2 * CPW])
            d0 = dstb0[pl.ds(i * L, L)]
            d1 = dstb1[pl.ds(i * L, L)]
            s0 = srcb0[pl.ds(i * L, L)]
            s1 = srcb1[pl.ds(i * L, L)]
            d = jnp.where(even, d0, d1)
            s = jnp.where(even, s0, s1)
            ld = d - b0
            plsc.addupdate_scatter(degl, [ld], ones,
                                   mask=(ld >= 0) & (ld < CPW * CH))
            for q in range(CPW):
                ldq = ld - q * CH
                mq = (ldq >= 0) & (ldq < CH)
                pkq = (ldq * PACK) | s
                posq = cs[q] + jnp.cumsum(mq.astype(jnp.int32)) - 1 + q * STG
                plsc.store_scatter(stage, [posq], pkq, mask=mq)
                cs[q] = cs[q] + plsc.all_reduce_population_count(mq)
                prq = jnp.any(cs[q] >= 256)

                @pl.when(prq)
                def _(q=q, fq=fs[q]):
                    pltpu.sync_copy(
                        stage.at[pl.ds(q * STG, 256)],
                        bkt_hbm.at[pl.ds((c0id + q) * EPADB + fq * 256, 256)])
                    stage[pl.ds(q * STG, L)] = stage[pl.ds(q * STG + 256, L)]

                fs[q] = fs[q] + prq.astype(jnp.int32)
                cs[q] = jnp.where(cs[q] >= 256, cs[q] - 256, cs[q])
            return tuple(cs) + tuple(fs)

        return lax.fori_loop(0, EBA // L, grp, carry, unroll=4)

    carry0 = (zc, zc, zc, zc, 0, 0, 0, 0)
    carry = lax.fori_loop(0, NBLKA, blk, carry0)
    for q in range(CPW):
        cq, fq = carry[q], carry[CPW + q]
        pltpu.sync_copy(stage.at[pl.ds(q * STG, 256)],
                        bkt_hbm.at[pl.ds((c0id + q) * EPADB + fq * 256, 256)])
        crow[pl.ds(0, L)] = fq * 256 + cq
        pltpu.sync_copy(crow, cnt_hbm.at[pl.ds((c0id + q) * L, L)])
    pltpu.sync_copy(degl, deg_hbm.at[pl.ds(b0, CPW * CH)])


@functools.partial(
    pl.kernel,
    mesh=_mesh(),
    compiler_params=_sc_params,
    out_type=jax.ShapeDtypeStruct((NPAD, F), jnp.float32),
    scratch_types=[
        pltpu.VMEM((CH + 1, F), jnp.float32),
        pltpu.VMEM((GW, F), jnp.float32),
        pltpu.VMEM((GW, F), jnp.float32),
        pltpu.VMEM((SB,), jnp.int32),
        pltpu.VMEM((GW,), jnp.int32),
        pltpu.VMEM((GW,), jnp.int32),
        pltpu.VMEM((L,), jnp.int32),
        pltpu.SemaphoreType.DMA,
        pltpu.SemaphoreType.DMA,
    ],
)
def _segmax_kernel(g_hbm, bkt_hbm, cnt_hbm, out_hbm,
                   acc, rows0, rows1, stage, fidx0, fidx1, crow, sem0, sem1):
    wid = lax.axis_index("s") * NC + lax.axis_index("c")
    lanes = lax.iota(jnp.int32, L)
    neg1 = jnp.full((L,), -1, jnp.int32)
    GPB = SB // GW

    def process(rows_ref, ld_a, ld_b):
        def row_body(j, c):
            jv = lanes * 0 + j
            sl = jnp.where(jv < L, ld_a, ld_b)
            lj = jnp.max(jnp.where(lanes == j % L, sl, neg1))
            for f0 in range(0, F // L, 8):
                avs = [acc[lj, pl.ds((f0 + k) * L, L)] for k in range(8)]
                rvs = [rows_ref[j, pl.ds((f0 + k) * L, L)] for k in range(8)]
                for k in range(8):
                    acc[lj, pl.ds((f0 + k) * L, L)] = jnp.maximum(avs[k], rvs[k])
            return c

        lax.fori_loop(0, GW, row_body, 0)

    for p in range(CPW):
        chunk = wid * CPW + p
        base = chunk * CH
        pltpu.sync_copy(g_hbm.at[pl.ds(base, CH)], acc.at[pl.ds(0, CH)])
        pltpu.sync_copy(cnt_hbm.at[pl.ds(chunk * L, L)], crow)
        total = jnp.max(crow[pl.ds(0, L)])
        ngrp = (total + GW - 1) // GW
        pltpu.sync_copy(bkt_hbm.at[pl.ds(chunk * EPADB, SB)], stage)

        def unpack(word_off, grp_idx):
            pka = stage[pl.ds(word_off, L)]
            pkb = stage[pl.ds(word_off + L, L)]
            va = (grp_idx * GW + lanes) < total
            vb = (grp_idx * GW + L + lanes) < total
            sa = jnp.where(va, pka & (PACK - 1), 0)
            sb = jnp.where(vb, pkb & (PACK - 1), 0)
            la = jnp.where(va, pka // PACK, CH)
            lb = jnp.where(vb, pkb // PACK, CH)
            return sa, sb, la, lb

        sa0, sb0, la0, lb0 = unpack(0, jnp.int32(0))

        @pl.when(ngrp > 0)
        def _():
            fidx0[pl.ds(0, L)] = sa0
            fidx0[pl.ds(L, L)] = sb0
            pltpu.async_copy(g_hbm.at[fidx0], rows0, sem0)

        def grp_body(gi, ld_cur):
            ld_a, ld_b = ld_cur
            nxt = gi + 1
            par = gi % 2

            @pl.when((nxt % GPB == 0) & (nxt < ngrp))
            def _():
                pltpu.sync_copy(
                    bkt_hbm.at[pl.ds(chunk * EPADB + (nxt // GPB) * SB, SB)],
                    stage)

            sa, sb, la, lb = unpack((nxt % GPB) * GW, nxt)

            @pl.when((par == 0) & (nxt < ngrp))
            def _():
                fidx1[pl.ds(0, L)] = sa
                fidx1[pl.ds(L, L)] = sb
                pltpu.async_copy(g_hbm.at[fidx1], rows1, sem1)

            @pl.when((par == 1) & (nxt < ngrp))
            def _():
                fidx0[pl.ds(0, L)] = sa
                fidx0[pl.ds(L, L)] = sb
                pltpu.async_copy(g_hbm.at[fidx0], rows0, sem0)

            @pl.when(par == 0)
            def _():
                pltpu.make_async_copy(g_hbm.at[fidx0], rows0, sem0).wait()
                process(rows0, ld_a, ld_b)

            @pl.when(par == 1)
            def _():
                pltpu.make_async_copy(g_hbm.at[fidx1], rows1, sem1).wait()
                process(rows1, ld_a, ld_b)

            return (la, lb)

        lax.fori_loop(0, ngrp, grp_body, (la0, lb0))
        pltpu.sync_copy(acc.at[pl.ds(0, CH)], out_hbm.at[pl.ds(base, CH)])


def _tc_g1_body(deg_ref, x_ref, w_ref, g_ref, dinv_ref):
    dinv = lax.rsqrt(jnp.maximum(deg_ref[...], 1e-12))
    dinv_ref[...] = dinv
    h = jnp.dot(x_ref[...], w_ref[...], preferred_element_type=jnp.float32)
    g_ref[...] = dinv * h


def _tc_mid_body(m_ref, dinv_ref, b1_ref, w2_ref, id_ref, g2_ref):
    dinv = dinv_ref[...]
    idv = jnp.maximum(dinv * m_ref[...] + b1_ref[...], 0.0)
    id_ref[...] = idv
    h2 = jnp.dot(idv, w2_ref[...], preferred_element_type=jnp.float32)
    g2_ref[...] = dinv * h2


def _tc_head_body(m_ref, dinv_ref, b2_ref, id_ref, w3a_ref, w3b_ref, b3_ref, o_ref):
    h2 = jnp.maximum(dinv_ref[...] * m_ref[...] + b2_ref[...], 0.0)
    o_ref[...] = (
        jnp.dot(h2, w3a_ref[...], preferred_element_type=jnp.float32)
        + jnp.dot(id_ref[...], w3b_ref[...], preferred_element_type=jnp.float32)
        + b3_ref[...]
    )


def kernel(x, edge_index, W1, b1, W2, b2, W3, b3):
    src = edge_index[0]
    dst = edge_index[1]
    srcp = jnp.pad(src, (0, EPAD - E))
    dstp = jnp.pad(dst, (0, EPAD - E), constant_values=-1)

    deg, bkt, cnts = _bucket_kernel(dstp, srcp)
    deg2 = deg.reshape(NPAD, 1)

    xp = jnp.pad(x, ((0, NPAD - N), (0, 8 - x.shape[1])))
    W1p = jnp.pad(W1, ((0, 8 - W1.shape[0]), (0, 0)))

    g1, dinv2 = pl.pallas_call(
        _tc_g1_body,
        grid=(NPAD // R,),
        in_specs=[
            pl.BlockSpec((R, 1), lambda i: (i, 0)),
            pl.BlockSpec((R, 8), lambda i: (i, 0)),
            pl.BlockSpec((8, F), lambda i: (0, 0)),
        ],
        out_specs=[
            pl.BlockSpec((R, F), lambda i: (i, 0)),
            pl.BlockSpec((R, 1), lambda i: (i, 0)),
        ],
        out_shape=[
            jax.ShapeDtypeStruct((NPAD, F), jnp.float32),
            jax.ShapeDtypeStruct((NPAD, 1), jnp.float32),
        ],
    )(deg2, xp, W1p)

    m1 = _segmax_kernel(g1, bkt, cnts)

    identity, g2 = pl.pallas_call(
        _tc_mid_body,
        grid=(NPAD // R,),
        in_specs=[
            pl.BlockSpec((R, F), lambda i: (i, 0)),
            pl.BlockSpec((R, 1), lambda i: (i, 0)),
            pl.BlockSpec((1, F), lambda i: (0, 0)),
            pl.BlockSpec((F, F), lambda i: (0, 0)),
        ],
        out_specs=[
            pl.BlockSpec((R, F), lambda i: (i, 0)),
            pl.BlockSpec((R, F), lambda i: (i, 0)),
        ],
        out_shape=[
            jax.ShapeDtypeStruct((NPAD, F), jnp.float32),
            jax.ShapeDtypeStruct((NPAD, F), jnp.float32),
        ],
    )(m1, dinv2, b1.reshape(1, F), W2)

    m2 = _segmax_kernel(g2, bkt, cnts)

    D_OUT = W3.shape[1]
    W3p = jnp.pad(W3, ((0, 0), (0, 128 - D_OUT)))
    b3p = jnp.pad(b3, (0, 128 - D_OUT)).reshape(1, 128)

    out = pl.pallas_call(
        _tc_head_body,
        grid=(NPAD // R,),
        in_specs=[
            pl.BlockSpec((R, F), lambda i: (i, 0)),
            pl.BlockSpec((R, 1), lambda i: (i, 0)),
            pl.BlockSpec((1, F), lambda i: (0, 0)),
            pl.BlockSpec((R, F), lambda i: (i, 0)),
            pl.BlockSpec((F, 128), lambda i: (0, 0)),
            pl.BlockSpec((F, 128), lambda i: (0, 0)),
            pl.BlockSpec((1, 128), lambda i: (0, 0)),
        ],
        out_specs=pl.BlockSpec((R, 128), lambda i: (i, 0)),
        out_shape=jax.ShapeDtypeStruct((NPAD, 128), jnp.float32),
    )(m2, dinv2, b2.reshape(1, F), identity, W3p[:F], W3p[F:], b3p)

    return out[:N, :D_OUT]


# speedup vs baseline: 2.3197x; 1.2475x over previous
import functools

import jax
import jax.numpy as jnp
from jax import lax
from jax.experimental import pallas as pl
from jax.experimental.pallas import tpu as pltpu
from jax.experimental.pallas import tpu_sc as plsc

N = 10000
NPAD = 10240
E = 320000
EBA = 2048
NBLKA = 157
EPAD = EBA * NBLKA
EPADB = EPAD + 256
F = 640
NC, NS, L = 2, 16, 16
NW = NC * NS
CH = 80
NCHUNK = NPAD // CH
CPW = NCHUNK // NW
GW = 32
SB = 1024
STG = 336
R = 512
PACK = 16384

_mesh = lambda: plsc.VectorSubcoreMesh(core_axis_name="c", subcore_axis_name="s")
_sc_params = pltpu.CompilerParams(needs_layout_passes=False)


@functools.partial(
    pl.kernel,
    mesh=_mesh(),
    compiler_params=_sc_params,
    out_type=[
        jax.ShapeDtypeStruct((NPAD,), jnp.float32),
        jax.ShapeDtypeStruct((NCHUNK * EPADB,), jnp.int32),
        jax.ShapeDtypeStruct((NCHUNK * L,), jnp.int32),
    ],
    scratch_types=[
        pltpu.VMEM((EBA,), jnp.int32),
        pltpu.VMEM((EBA,), jnp.int32),
        pltpu.VMEM((EBA,), jnp.int32),
        pltpu.VMEM((EBA,), jnp.int32),
        pltpu.VMEM((CPW * CH,), jnp.float32),
        pltpu.VMEM((CPW * STG,), jnp.int32),
        pltpu.VMEM((L,), jnp.int32),
        pltpu.SemaphoreType.DMA,
        pltpu.SemaphoreType.DMA,
    ],
)
def _bucket_kernel(dst_hbm, src_hbm, deg_hbm, bkt_hbm, cnt_hbm,
                   dstb0, srcb0, dstb1, srcb1, degl, stage, crow, semA, semB):
    wid = lax.axis_index("s") * NC + lax.axis_index("c")
    c0id = wid * CPW
    b0 = c0id * CH
    ones = jnp.full((L,), 1.0, jnp.float32)
    zc = jnp.zeros((L,), jnp.int32)

    def init(i, c):
        degl[pl.ds(i * L, L)] = ones
        return c

    lax.fori_loop(0, CPW * CH // L, init, 0)

    def issue(b):
        @pl.when((b % 2 == 0) & (b < NBLKA))
        def _():
            pltpu.async_copy(dst_hbm.at[pl.ds(b * EBA, EBA)], dstb0, semA)
            pltpu.async_copy(src_hbm.at[pl.ds(b * EBA, EBA)], srcb0, semA)

        @pl.when((b % 2 == 1) & (b < NBLKA))
        def _():
            pltpu.async_copy(dst_hbm.at[pl.ds(b * EBA, EBA)], dstb1, semB)
            pltpu.async_copy(src_hbm.at[pl.ds(b * EBA, EBA)], srcb1, semB)

    def waitblk(b):
        @pl.when(b % 2 == 0)
        def _():
            pltpu.make_async_copy(dst_hbm.at[pl.ds(0, EBA)], dstb0, semA).wait()
            pltpu.make_async_copy(src_hbm.at[pl.ds(0, EBA)], srcb0, semA).wait()

        @pl.when(b % 2 == 1)
        def _():
            pltpu.make_async_copy(dst_hbm.at[pl.ds(0, EBA)], dstb1, semB).wait()
            pltpu.make_async_copy(src_hbm.at[pl.ds(0, EBA)], srcb1, semB).wait()

    issue(jnp.int32(0))

    def blk(b, carry):
        issue(b + 1)
        waitblk(b)
        even = (b % 2) == 0

        def grp4(o, carry2):
            cs = list(carry2[0:CPW])
            fs = list(carry2[CPW:2 * CPW])
            for u in range(4):
                i = o * 4 + u
                d0 = dstb0[pl.ds(i * L, L)]
                d1 = dstb1[pl.ds(i * L, L)]
                s0 = srcb0[pl.ds(i * L, L)]
                s1 = srcb1[pl.ds(i * L, L)]
                d = jnp.where(even, d0, d1)
                s = jnp.where(even, s0, s1)
                ld = d - b0
                plsc.addupdate_scatter(degl, [ld], ones,
                                       mask=(ld >= 0) & (ld < CPW * CH))
                for q in range(CPW):
                    ldq = ld - q * CH
                    mq = (ldq >= 0) & (ldq < CH)
                    pkq = (ldq * PACK) | s
                    posq = (cs[q] + jnp.cumsum(mq.astype(jnp.int32)) - 1
                            + q * STG)
                    plsc.store_scatter(stage, [posq], pkq, mask=mq)
                    cs[q] = cs[q] + plsc.all_reduce_population_count(mq)
            for q in range(CPW):
                prq = jnp.any(cs[q] >= 256)

                @pl.when(prq)
                def _(q=q, fq=fs[q]):
                    pltpu.sync_copy(
                        stage.at[pl.ds(q * STG, 256)],
                        bkt_hbm.at[pl.ds((c0id + q) * EPADB + fq * 256, 256)])
                    for t in range(4):
                        stage[pl.ds(q * STG + t * L, L)] = (
                            stage[pl.ds(q * STG + 256 + t * L, L)])

                fs[q] = fs[q] + prq.astype(jnp.int32)
                cs[q] = jnp.where(cs[q] >= 256, cs[q] - 256, cs[q])
            return tuple(cs) + tuple(fs)

        return lax.fori_loop(0, EBA // L // 4, grp4, carry)

    carry0 = (zc, zc, zc, zc, 0, 0, 0, 0)
    carry = lax.fori_loop(0, NBLKA, blk, carry0)
    for q in range(CPW):
        cq, fq = carry[q], carry[CPW + q]
        pltpu.sync_copy(stage.at[pl.ds(q * STG, 256)],
                        bkt_hbm.at[pl.ds((c0id + q) * EPADB + fq * 256, 256)])
        crow[pl.ds(0, L)] = fq * 256 + cq
        pltpu.sync_copy(crow, cnt_hbm.at[pl.ds((c0id + q) * L, L)])
    pltpu.sync_copy(degl, deg_hbm.at[pl.ds(b0, CPW * CH)])


@functools.partial(
    pl.kernel,
    mesh=_mesh(),
    compiler_params=_sc_params,
    out_type=jax.ShapeDtypeStruct((NPAD, F), jnp.float32),
    scratch_types=[
        pltpu.VMEM((CH + 1, F), jnp.float32),
        pltpu.VMEM((GW, F), jnp.float32),
        pltpu.VMEM((GW, F), jnp.float32),
        pltpu.VMEM((SB,), jnp.int32),
        pltpu.VMEM((GW,), jnp.int32),
        pltpu.VMEM((GW,), jnp.int32),
        pltpu.VMEM((L,), jnp.int32),
        pltpu.SemaphoreType.DMA,
        pltpu.SemaphoreType.DMA,
    ],
)
def _segmax_kernel(g_hbm, bkt_hbm, cnt_hbm, out_hbm,
                   acc, rows0, rows1, stage, fidx0, fidx1, crow, sem0, sem1):
    wid = lax.axis_index("s") * NC + lax.axis_index("c")
    lanes = lax.iota(jnp.int32, L)
    neg1 = jnp.full((L,), -1, jnp.int32)
    GPB = SB // GW

    def process(rows_ref, ld_a, ld_b):
        def row_body(j, c):
            jv = lanes * 0 + j
            sl = jnp.where(jv < L, ld_a, ld_b)
            lj = jnp.max(jnp.where(lanes == j % L, sl, neg1))
            for f0 in range(0, F // L, 8):
                avs = [acc[lj, pl.ds((f0 + k) * L, L)] for k in range(8)]
                rvs = [rows_ref[j, pl.ds((f0 + k) * L, L)] for k in range(8)]
                for k in range(8):
                    acc[lj, pl.ds((f0 + k) * L, L)] = jnp.maximum(avs[k], rvs[k])
            return c

        lax.fori_loop(0, GW, row_body, 0)

    for p in range(CPW):
        chunk = wid * CPW + p
        base = chunk * CH
        pltpu.sync_copy(g_hbm.at[pl.ds(base, CH)], acc.at[pl.ds(0, CH)])
        pltpu.sync_copy(cnt_hbm.at[pl.ds(chunk * L, L)], crow)
        total = jnp.max(crow[pl.ds(0, L)])
        ngrp = (total + GW - 1) // GW
        pltpu.sync_copy(bkt_hbm.at[pl.ds(chunk * EPADB, SB)], stage)

        def unpack(word_off, grp_idx):
            pka = stage[pl.ds(word_off, L)]
            pkb = stage[pl.ds(word_off + L, L)]
            va = (grp_idx * GW + lanes) < total
            vb = (grp_idx * GW + L + lanes) < total
            sa = jnp.where(va, pka & (PACK - 1), 0)
            sb = jnp.where(vb, pkb & (PACK - 1), 0)
            la = jnp.where(va, pka // PACK, CH)
            lb = jnp.where(vb, pkb // PACK, CH)
            return sa, sb, la, lb

        sa0, sb0, la0, lb0 = unpack(0, jnp.int32(0))

        @pl.when(ngrp > 0)
        def _():
            fidx0[pl.ds(0, L)] = sa0
            fidx0[pl.ds(L, L)] = sb0
            pltpu.async_copy(g_hbm.at[fidx0], rows0, sem0)

        def grp_body(gi, ld_cur):
            ld_a, ld_b = ld_cur
            nxt = gi + 1
            par = gi % 2

            @pl.when((nxt % GPB == 0) & (nxt < ngrp))
            def _():
                pltpu.sync_copy(
                    bkt_hbm.at[pl.ds(chunk * EPADB + (nxt // GPB) * SB, SB)],
                    stage)

            sa, sb, la, lb = unpack((nxt % GPB) * GW, nxt)

            @pl.when((par == 0) & (nxt < ngrp))
            def _():
                fidx1[pl.ds(0, L)] = sa
                fidx1[pl.ds(L, L)] = sb
                pltpu.async_copy(g_hbm.at[fidx1], rows1, sem1)

            @pl.when((par == 1) & (nxt < ngrp))
            def _():
                fidx0[pl.ds(0, L)] = sa
                fidx0[pl.ds(L, L)] = sb
                pltpu.async_copy(g_hbm.at[fidx0], rows0, sem0)

            @pl.when(par == 0)
            def _():
                pltpu.make_async_copy(g_hbm.at[fidx0], rows0, sem0).wait()
                process(rows0, ld_a, ld_b)

            @pl.when(par == 1)
            def _():
                pltpu.make_async_copy(g_hbm.at[fidx1], rows1, sem1).wait()
                process(rows1, ld_a, ld_b)

            return (la, lb)

        lax.fori_loop(0, ngrp, grp_body, (la0, lb0))
        pltpu.sync_copy(acc.at[pl.ds(0, CH)], out_hbm.at[pl.ds(base, CH)])


def _tc_g1_body(deg_ref, x_ref, w_ref, g_ref, dinv_ref):
    dinv = lax.rsqrt(jnp.maximum(deg_ref[...], 1e-12))
    dinv_ref[...] = dinv
    h = jnp.dot(x_ref[...], w_ref[...], preferred_element_type=jnp.float32)
    g_ref[...] = dinv * h


def _tc_mid_body(m_ref, dinv_ref, b1_ref, w2_ref, id_ref, g2_ref):
    dinv = dinv_ref[...]
    idv = jnp.maximum(dinv * m_ref[...] + b1_ref[...], 0.0)
    id_ref[...] = idv
    h2 = jnp.dot(idv, w2_ref[...], preferred_element_type=jnp.float32)
    g2_ref[...] = dinv * h2


def _tc_head_body(m_ref, dinv_ref, b2_ref, id_ref, w3a_ref, w3b_ref, b3_ref, o_ref):
    h2 = jnp.maximum(dinv_ref[...] * m_ref[...] + b2_ref[...], 0.0)
    o_ref[...] = (
        jnp.dot(h2, w3a_ref[...], preferred_element_type=jnp.float32)
        + jnp.dot(id_ref[...], w3b_ref[...], preferred_element_type=jnp.float32)
        + b3_ref[...]
    )


def kernel(x, edge_index, W1, b1, W2, b2, W3, b3):
    src = edge_index[0]
    dst = edge_index[1]
    srcp = jnp.pad(src, (0, EPAD - E))
    dstp = jnp.pad(dst, (0, EPAD - E), constant_values=-1)

    deg, bkt, cnts = _bucket_kernel(dstp, srcp)
    deg2 = deg.reshape(NPAD, 1)

    xp = jnp.pad(x, ((0, NPAD - N), (0, 8 - x.shape[1])))
    W1p = jnp.pad(W1, ((0, 8 - W1.shape[0]), (0, 0)))

    g1, dinv2 = pl.pallas_call(
        _tc_g1_body,
        grid=(NPAD // R,),
        in_specs=[
            pl.BlockSpec((R, 1), lambda i: (i, 0)),
            pl.BlockSpec((R, 8), lambda i: (i, 0)),
            pl.BlockSpec((8, F), lambda i: (0, 0)),
        ],
        out_specs=[
            pl.BlockSpec((R, F), lambda i: (i, 0)),
            pl.BlockSpec((R, 1), lambda i: (i, 0)),
        ],
        out_shape=[
            jax.ShapeDtypeStruct((NPAD, F), jnp.float32),
            jax.ShapeDtypeStruct((NPAD, 1), jnp.float32),
        ],
    )(deg2, xp, W1p)

    m1 = _segmax_kernel(g1, bkt, cnts)

    identity, g2 = pl.pallas_call(
        _tc_mid_body,
        grid=(NPAD // R,),
        in_specs=[
            pl.BlockSpec((R, F), lambda i: (i, 0)),
            pl.BlockSpec((R, 1), lambda i: (i, 0)),
            pl.BlockSpec((1, F), lambda i: (0, 0)),
            pl.BlockSpec((F, F), lambda i: (0, 0)),
        ],
        out_specs=[
            pl.BlockSpec((R, F), lambda i: (i, 0)),
            pl.BlockSpec((R, F), lambda i: (i, 0)),
        ],
        out_shape=[
            jax.ShapeDtypeStruct((NPAD, F), jnp.float32),
            jax.ShapeDtypeStruct((NPAD, F), jnp.float32),
        ],
    )(m1, dinv2, b1.reshape(1, F), W2)

    m2 = _segmax_kernel(g2, bkt, cnts)

    D_OUT = W3.shape[1]
    W3p = jnp.pad(W3, ((0, 0), (0, 128 - D_OUT)))
    b3p = jnp.pad(b3, (0, 128 - D_OUT)).reshape(1, 128)

    out = pl.pallas_call(
        _tc_head_body,
        grid=(NPAD // R,),
        in_specs=[
            pl.BlockSpec((R, F), lambda i: (i, 0)),
            pl.BlockSpec((R, 1), lambda i: (i, 0)),
            pl.BlockSpec((1, F), lambda i: (0, 0)),
            pl.BlockSpec((R, F), lambda i: (i, 0)),
            pl.BlockSpec((F, 128), lambda i: (0, 0)),
            pl.BlockSpec((F, 128), lambda i: (0, 0)),
            pl.BlockSpec((1, 128), lambda i: (0, 0)),
        ],
        out_specs=pl.BlockSpec((R, 128), lambda i: (i, 0)),
        out_shape=jax.ShapeDtypeStruct((NPAD, 128), jnp.float32),
    )(m2, dinv2, b2.reshape(1, F), identity, W3p[:F], W3p[F:], b3p)

    return out[:N, :D_OUT]
